# baseline (speedup 1.0000x reference)
import functools

import jax
import jax.numpy as jnp
from jax import lax
from jax.experimental import pallas as pl
from jax.experimental.pallas import tpu as pltpu
from jax.experimental.pallas import tpu_sc as plsc

NC = 2
NS = 16
L = 16


def _edge_mm_body(ea_ref, we_ref, be_ref, out_ref):
    out_ref[...] = (
        jnp.dot(ea_ref[...], we_ref[...], preferred_element_type=jnp.float32)
        + be_ref[...])


def _edge_mm(edge_attr, We, be):
    E, ED = edge_attr.shape
    D = We.shape[1]
    BLK = 2000
    assert E % BLK == 0
    return pl.pallas_call(
        _edge_mm_body,
        grid=(E // BLK,),
        in_specs=[
            pl.BlockSpec((BLK, ED), lambda i: (i, 0)),
            pl.BlockSpec((ED, D), lambda i: (0, 0)),
            pl.BlockSpec((1, D), lambda i: (0, 0)),
        ],
        out_specs=pl.BlockSpec((BLK, D), lambda i: (i, 0)),
        out_shape=jax.ShapeDtypeStruct((E, D), jnp.float32),
    )(edge_attr, We, be.reshape(1, D))


def _make_sc_scatter(N, D, E):
    NW = NC * NS
    EPW = E // NW
    C = 80
    assert EPW % C == 0
    NCHUNK = EPW // C
    RPT0 = (N // NS) // 8 * 8
    RPT_LAST = N - (NS - 1) * RPT0
    assert RPT_LAST % 8 == 0
    SL2 = D // (2 * L)

    mesh = plsc.VectorSubcoreMesh(core_axis_name="c", subcore_axis_name="s")

    @functools.partial(
        pl.kernel,
        mesh=mesh,
        out_type=jax.ShapeDtypeStruct((NC, N, D), jnp.float32),
        scratch_types=[
            pltpu.VMEM((2, C), jnp.int32),
            pltpu.VMEM((2, C), jnp.int32),
            pltpu.VMEM((2, C, D), jnp.float32),
            pltpu.VMEM((2, C, D), jnp.float32),
            pltpu.VMEM_SHARED((N, D), jnp.float32),
            pltpu.SemaphoreType.DMA((2,)),
            pltpu.SemaphoreType.DMA((2,)),
            pltpu.SemaphoreType.DMA((2,)),
        ],
    )
    def sc_scatter(zeros_hbm, x_hbm, src_hbm, dst_hbm, rea_hbm, out_hbm,
                   srcb, dstb, rows2, ea2, acc_sh, gsem, esem, isem):
        c = lax.axis_index("c")
        s = lax.axis_index("s")
        w = c * NS + s
        base = w * EPW

        def start_sidx(j, b):
            pltpu.async_copy(src_hbm.at[w, j], srcb.at[b], isem.at[b])

        def wait_sidx(j, b):
            pltpu.make_async_copy(src_hbm.at[w, j], srcb.at[b],
                                  isem.at[b]).wait()

        def start_main(j, b):
            pltpu.async_copy(rea_hbm.at[pl.ds(base + j * C, C)], ea2.at[b],
                             esem.at[b])
            pltpu.async_copy(dst_hbm.at[w, j], dstb.at[b], esem.at[b])
            pltpu.async_copy(x_hbm.at[srcb.at[b]], rows2.at[b], gsem.at[b])

        def wait_main(j, b):
            pltpu.make_async_copy(rea_hbm.at[pl.ds(base + j * C, C)],
                                  ea2.at[b], esem.at[b]).wait()
            pltpu.make_async_copy(dst_hbm.at[w, j], dstb.at[b],
                                  esem.at[b]).wait()
            pltpu.make_async_copy(x_hbm.at[srcb.at[b]], rows2.at[b],
                                  gsem.at[b]).wait()

        start_sidx(0, 0)
        start_sidx(1, 1)

        @pl.when(s < NS - 1)
        def _():
            pltpu.sync_copy(zeros_hbm.at[pl.ds(s * RPT0, RPT0)],
                            acc_sh.at[pl.ds(s * RPT0, RPT0)])

        @pl.when(s == NS - 1)
        def _():
            pltpu.sync_copy(zeros_hbm.at[pl.ds(s * RPT0, RPT_LAST)],
                            acc_sh.at[pl.ds(s * RPT0, RPT_LAST)])

        plsc.subcore_barrier()

        wait_sidx(0, 0)
        start_main(0, 0)

        def chunk_body(i, carry):
            b = lax.rem(i, 2)
            nb = 1 - b

            @pl.when(i + 1 < NCHUNK)
            def _():
                wait_sidx(i + 1, nb)
                start_main(i + 1, nb)

            wait_main(i, b)

            @pl.when(i + 2 < NCHUNK)
            def _():
                start_sidx(i + 2, b)

            rows_b = rows2.at[b]
            ea_b = ea2.at[b]

            @plsc.parallel_loop(0, C, unroll=4)
            def _(j):
                for k in range(D // L):
                    sl = (j, pl.ds(k * L, L))
                    rows_b[sl] = jnp.maximum(rows_b[sl] + ea_b[sl], 0.0)

            pltpu.sync_copy(rows2.at[b], acc_sh.at[dstb.at[b]], add=True)
            return carry

        lax.fori_loop(0, NCHUNK, chunk_body, 0)
        plsc.subcore_barrier()

        @pl.when(s < NS - 1)
        def _():
            pltpu.sync_copy(acc_sh.at[pl.ds(s * RPT0, RPT0)],
                            out_hbm.at[c, pl.ds(s * RPT0, RPT0)])

        @pl.when(s == NS - 1)
        def _():
            pltpu.sync_copy(acc_sh.at[pl.ds(s * RPT0, RPT_LAST)],
                            out_hbm.at[c, pl.ds(s * RPT0, RPT_LAST)])

    return sc_scatter


def _mlp_body(scale_ref, x_ref, p_ref, w1_ref, b1_ref, g_ref, bt_ref,
              w2_ref, b2_ref, out_ref):
    h = x_ref[...] * scale_ref[0] + p_ref[0] + p_ref[1]
    h = jnp.dot(h, w1_ref[...], preferred_element_type=jnp.float32) + b1_ref[...]
    mu = jnp.mean(h, axis=-1, keepdims=True)
    var = jnp.mean((h - mu) * (h - mu), axis=-1, keepdims=True)
    h = (h - mu) * lax.rsqrt(var + 1e-5) * g_ref[...] + bt_ref[...]
    h = jnp.maximum(h, 0.0)
    out_ref[...] = (
        jnp.dot(h, w2_ref[...], preferred_element_type=jnp.float32) + b2_ref[...]
    )


def _mlp(scale, x, partials, W1, b1, gamma, beta, W2, b2):
    N, D = x.shape
    H = W1.shape[1]
    BLK = 2000
    assert N % BLK == 0
    return pl.pallas_call(
        _mlp_body,
        grid=(N // BLK,),
        in_specs=[
            pl.BlockSpec(memory_space=pltpu.SMEM),
            pl.BlockSpec((BLK, D), lambda i: (i, 0)),
            pl.BlockSpec((NC, BLK, D), lambda i: (0, i, 0)),
            pl.BlockSpec((D, H), lambda i: (0, 0)),
            pl.BlockSpec((1, H), lambda i: (0, 0)),
            pl.BlockSpec((1, H), lambda i: (0, 0)),
            pl.BlockSpec((1, H), lambda i: (0, 0)),
            pl.BlockSpec((H, D), lambda i: (0, 0)),
            pl.BlockSpec((1, D), lambda i: (0, 0)),
        ],
        out_specs=pl.BlockSpec((BLK, D), lambda i: (i, 0)),
        out_shape=jax.ShapeDtypeStruct((N, D), jnp.float32),
    )(scale, x, partials, W1, b1.reshape(1, H), gamma.reshape(1, H),
      beta.reshape(1, H), W2, b2.reshape(1, D))


def kernel(x, edge_index, edge_attr, We, be, W1, b1, gamma, beta, W2, b2, eps):
    N, D = x.shape
    E = edge_attr.shape[0]

    rea = _edge_mm(edge_attr, We, be)

    NW = NC * NS
    C = 80
    src = edge_index[1].astype(jnp.int32).reshape(NW, E // (NW * C), C)
    dst = edge_index[0].astype(jnp.int32).reshape(NW, E // (NW * C), C)
    zeros = jnp.zeros((N, D), dtype=jnp.float32)
    partials = _make_sc_scatter(N, D, E)(zeros, x, src, dst, rea)

    scale = (1.0 + eps).astype(jnp.float32).reshape(1)
    return _mlp(scale, x, partials, W1, b1, gamma, beta, W2, b2)


# speedup vs baseline: 1.1368x; 1.1368x over previous
import functools

import jax
import jax.numpy as jnp
from jax import lax
from jax.experimental import pallas as pl
from jax.experimental.pallas import tpu as pltpu
from jax.experimental.pallas import tpu_sc as plsc

NC = 2
NS = 16
L = 16


def _edge_mm_body(ea_ref, we_ref, be_ref, out_ref):
    out_ref[...] = (
        jnp.dot(ea_ref[...], we_ref[...], preferred_element_type=jnp.float32)
        + be_ref[...])


def _edge_mm(edge_attr, We, be, row0, nrows):
    ED = edge_attr.shape[1]
    D = We.shape[1]
    BLK = 2560
    assert row0 % BLK == 0 and nrows % BLK == 0
    blk0 = row0 // BLK
    return pl.pallas_call(
        _edge_mm_body,
        grid=(nrows // BLK,),
        in_specs=[
            pl.BlockSpec((BLK, ED), lambda i: (i + blk0, 0)),
            pl.BlockSpec((ED, D), lambda i: (0, 0)),
            pl.BlockSpec((1, D), lambda i: (0, 0)),
        ],
        out_specs=pl.BlockSpec((BLK, D), lambda i: (i, 0)),
        out_shape=jax.ShapeDtypeStruct((nrows, D), jnp.float32),
    )(edge_attr, We, be.reshape(1, D))


def _make_sc_scatter(N, D, NCHUNK, chained):
    NW = NC * NS
    C = 80
    EPW = NCHUNK * C
    RPT0 = (N // NS) // 8 * 8
    RPT_LAST = N - (NS - 1) * RPT0
    assert RPT_LAST % 8 == 0
    SL2 = D // (2 * L)

    mesh = plsc.VectorSubcoreMesh(core_axis_name="c", subcore_axis_name="s")

    @functools.partial(
        pl.kernel,
        mesh=mesh,
        out_type=jax.ShapeDtypeStruct((NC, N, D), jnp.float32),
        scratch_types=[
            pltpu.VMEM((2, C), jnp.int32),
            pltpu.VMEM((2, C), jnp.int32),
            pltpu.VMEM((2, C, D), jnp.float32),
            pltpu.VMEM((2, C, D), jnp.float32),
            pltpu.VMEM_SHARED((N, D), jnp.float32),
            pltpu.SemaphoreType.DMA((2,)),
            pltpu.SemaphoreType.DMA((2,)),
            pltpu.SemaphoreType.DMA((2,)),
        ],
    )
    def sc_scatter(init_hbm, x_hbm, src_hbm, dst_hbm, rea_hbm, out_hbm,
                   srcb, dstb, rows2, ea2, acc_sh, gsem, esem, isem):
        c = lax.axis_index("c")
        s = lax.axis_index("s")
        w = c * NS + s
        base = w * EPW

        def start_sidx(j, b):
            pltpu.async_copy(src_hbm.at[w, j], srcb.at[b], isem.at[b])

        def wait_sidx(j, b):
            pltpu.make_async_copy(src_hbm.at[w, j], srcb.at[b],
                                  isem.at[b]).wait()

        def start_main(j, b):
            pltpu.async_copy(rea_hbm.at[pl.ds(base + j * C, C)], ea2.at[b],
                             esem.at[b])
            pltpu.async_copy(dst_hbm.at[w, j], dstb.at[b], esem.at[b])
            pltpu.async_copy(x_hbm.at[srcb.at[b]], rows2.at[b], gsem.at[b])

        def wait_main(j, b):
            pltpu.make_async_copy(rea_hbm.at[pl.ds(base + j * C, C)],
                                  ea2.at[b], esem.at[b]).wait()
            pltpu.make_async_copy(dst_hbm.at[w, j], dstb.at[b],
                                  esem.at[b]).wait()
            pltpu.make_async_copy(x_hbm.at[srcb.at[b]], rows2.at[b],
                                  gsem.at[b]).wait()

        start_sidx(0, 0)
        start_sidx(1, 1)

        def _init_src(r0, nr):
            if chained:
                return init_hbm.at[c, pl.ds(r0, nr)]
            return init_hbm.at[pl.ds(r0, nr)]

        @pl.when(s < NS - 1)
        def _():
            pltpu.sync_copy(_init_src(s * RPT0, RPT0),
                            acc_sh.at[pl.ds(s * RPT0, RPT0)])

        @pl.when(s == NS - 1)
        def _():
            pltpu.sync_copy(_init_src(s * RPT0, RPT_LAST),
                            acc_sh.at[pl.ds(s * RPT0, RPT_LAST)])

        plsc.subcore_barrier()

        wait_sidx(0, 0)
        start_main(0, 0)

        def chunk_body(i, carry):
            b = lax.rem(i, 2)
            nb = 1 - b

            @pl.when(i + 1 < NCHUNK)
            def _():
                wait_sidx(i + 1, nb)
                start_main(i + 1, nb)

            wait_main(i, b)

            @pl.when(i + 2 < NCHUNK)
            def _():
                start_sidx(i + 2, b)

            rows_b = rows2.at[b]
            ea_b = ea2.at[b]

            @plsc.parallel_loop(0, C, unroll=4)
            def _(j):
                for k in range(D // L):
                    sl = (j, pl.ds(k * L, L))
                    rows_b[sl] = jnp.maximum(rows_b[sl] + ea_b[sl], 0.0)

            pltpu.sync_copy(rows2.at[b], acc_sh.at[dstb.at[b]], add=True)
            return carry

        lax.fori_loop(0, NCHUNK, chunk_body, 0)
        plsc.subcore_barrier()

        @pl.when(s < NS - 1)
        def _():
            pltpu.sync_copy(acc_sh.at[pl.ds(s * RPT0, RPT0)],
                            out_hbm.at[c, pl.ds(s * RPT0, RPT0)])

        @pl.when(s == NS - 1)
        def _():
            pltpu.sync_copy(acc_sh.at[pl.ds(s * RPT0, RPT_LAST)],
                            out_hbm.at[c, pl.ds(s * RPT0, RPT_LAST)])

    return sc_scatter


def _mlp_body(scale_ref, x_ref, p_ref, w1_ref, b1_ref, g_ref, bt_ref,
              w2_ref, b2_ref, out_ref):
    h = x_ref[...] * scale_ref[0] + p_ref[0] + p_ref[1]
    h = jnp.dot(h, w1_ref[...], preferred_element_type=jnp.float32) + b1_ref[...]
    mu = jnp.mean(h, axis=-1, keepdims=True)
    var = jnp.mean((h - mu) * (h - mu), axis=-1, keepdims=True)
    h = (h - mu) * lax.rsqrt(var + 1e-5) * g_ref[...] + bt_ref[...]
    h = jnp.maximum(h, 0.0)
    out_ref[...] = (
        jnp.dot(h, w2_ref[...], preferred_element_type=jnp.float32) + b2_ref[...]
    )


def _mlp(scale, x, partials, W1, b1, gamma, beta, W2, b2):
    N, D = x.shape
    H = W1.shape[1]
    BLK = 2000
    assert N % BLK == 0
    return pl.pallas_call(
        _mlp_body,
        grid=(N // BLK,),
        in_specs=[
            pl.BlockSpec(memory_space=pltpu.SMEM),
            pl.BlockSpec((BLK, D), lambda i: (i, 0)),
            pl.BlockSpec((NC, BLK, D), lambda i: (0, i, 0)),
            pl.BlockSpec((D, H), lambda i: (0, 0)),
            pl.BlockSpec((1, H), lambda i: (0, 0)),
            pl.BlockSpec((1, H), lambda i: (0, 0)),
            pl.BlockSpec((1, H), lambda i: (0, 0)),
            pl.BlockSpec((H, D), lambda i: (0, 0)),
            pl.BlockSpec((1, D), lambda i: (0, 0)),
        ],
        out_specs=pl.BlockSpec((BLK, D), lambda i: (i, 0)),
        out_shape=jax.ShapeDtypeStruct((N, D), jnp.float32),
    )(scale, x, partials, W1, b1.reshape(1, H), gamma.reshape(1, H),
      beta.reshape(1, H), W2, b2.reshape(1, D))


def kernel(x, edge_index, edge_attr, We, be, W1, b1, gamma, beta, W2, b2, eps):
    N, D = x.shape
    E = edge_attr.shape[0]

    NW = NC * NS
    C = 80
    NCH0, NCH1 = 62, 63
    E0 = NW * NCH0 * C
    assert E0 % 2560 == 0 and (E - E0) % 2560 == 0

    rea0 = _edge_mm(edge_attr, We, be, 0, E0)
    rea1 = _edge_mm(edge_attr, We, be, E0, E - E0)

    src = edge_index[1].astype(jnp.int32)
    dst = edge_index[0].astype(jnp.int32)
    src0 = src[:E0].reshape(NW, NCH0, C)
    dst0 = dst[:E0].reshape(NW, NCH0, C)
    src1 = src[E0:].reshape(NW, NCH1, C)
    dst1 = dst[E0:].reshape(NW, NCH1, C)
    zeros = jnp.zeros((N, D), dtype=jnp.float32)
    p0 = _make_sc_scatter(N, D, NCH0, False)(zeros, x, src0, dst0, rea0)
    partials = _make_sc_scatter(N, D, NCH1, True)(p0, x, src1, dst1, rea1)

    scale = (1.0 + eps).astype(jnp.float32).reshape(1)
    return _mlp(scale, x, partials, W1, b1, gamma, beta, W2, b2)


# speedup vs baseline: 1.5314x; 1.3471x over previous
import functools

import jax
import jax.numpy as jnp
from jax import lax
from jax.experimental import pallas as pl
from jax.experimental.pallas import tpu as pltpu
from jax.experimental.pallas import tpu_sc as plsc

NC = 2
NS = 16
L = 16


def _edge_mm_body(ea_ref, we_ref, be_ref, out_ref):
    out_ref[...] = lax.dot_general(
        ea_ref[...], we_ref[...], (((0,), (0,)), ((), ())),
        preferred_element_type=jnp.float32) + be_ref[...]


def _edge_mm(edge_attr_t, We, be, row0, nrows):
    ED = edge_attr_t.shape[0]
    D = We.shape[1]
    BLK = 2560
    assert row0 % BLK == 0 and nrows % BLK == 0
    blk0 = row0 // BLK
    return pl.pallas_call(
        _edge_mm_body,
        grid=(nrows // BLK,),
        in_specs=[
            pl.BlockSpec((ED, BLK), lambda i: (0, i + blk0)),
            pl.BlockSpec((ED, D), lambda i: (0, 0)),
            pl.BlockSpec((1, D), lambda i: (0, 0)),
        ],
        out_specs=pl.BlockSpec((BLK, D), lambda i: (i, 0)),
        out_shape=jax.ShapeDtypeStruct((nrows, D), jnp.float32),
    )(edge_attr_t, We, be.reshape(1, D))


def _make_sc_scatter(N, D, NCHUNK, chained):
    NW = NC * NS
    C = 80
    EPW = NCHUNK * C
    RPT0 = (N // NS) // 8 * 8
    RPT_LAST = N - (NS - 1) * RPT0
    assert RPT_LAST % 8 == 0
    SL2 = D // (2 * L)

    mesh = plsc.VectorSubcoreMesh(core_axis_name="c", subcore_axis_name="s")

    @functools.partial(
        pl.kernel,
        mesh=mesh,
        out_type=jax.ShapeDtypeStruct((NC, N, D), jnp.float32),
        scratch_types=[
            pltpu.VMEM((2, C), jnp.int32),
            pltpu.VMEM((2, C), jnp.int32),
            pltpu.VMEM((2, C, D), jnp.float32),
            pltpu.VMEM((2, C, D), jnp.float32),
            pltpu.VMEM_SHARED((N, D), jnp.float32),
            pltpu.SemaphoreType.DMA((2,)),
            pltpu.SemaphoreType.DMA((2,)),
            pltpu.SemaphoreType.DMA((2,)),
        ],
    )
    def sc_scatter(init_hbm, x_hbm, src_hbm, dst_hbm, rea_hbm, out_hbm,
                   srcb, dstb, rows2, ea2, acc_sh, gsem, esem, isem):
        c = lax.axis_index("c")
        s = lax.axis_index("s")
        w = c * NS + s
        base = w * EPW

        def start_sidx(j, b):
            pltpu.async_copy(src_hbm.at[w, j], srcb.at[b], isem.at[b])

        def wait_sidx(j, b):
            pltpu.make_async_copy(src_hbm.at[w, j], srcb.at[b],
                                  isem.at[b]).wait()

        def start_main(j, b):
            pltpu.async_copy(rea_hbm.at[pl.ds(base + j * C, C)], ea2.at[b],
                             esem.at[b])
            pltpu.async_copy(dst_hbm.at[w, j], dstb.at[b], esem.at[b])
            pltpu.async_copy(x_hbm.at[srcb.at[b]], rows2.at[b], gsem.at[b])

        def wait_main(j, b):
            pltpu.make_async_copy(rea_hbm.at[pl.ds(base + j * C, C)],
                                  ea2.at[b], esem.at[b]).wait()
            pltpu.make_async_copy(dst_hbm.at[w, j], dstb.at[b],
                                  esem.at[b]).wait()
            pltpu.make_async_copy(x_hbm.at[srcb.at[b]], rows2.at[b],
                                  gsem.at[b]).wait()

        start_sidx(0, 0)
        start_sidx(1, 1)

        def _init_src(r0, nr):
            if chained:
                return init_hbm.at[c, pl.ds(r0, nr)]
            return init_hbm.at[pl.ds(r0, nr)]

        @pl.when(s < NS - 1)
        def _():
            pltpu.sync_copy(_init_src(s * RPT0, RPT0),
                            acc_sh.at[pl.ds(s * RPT0, RPT0)])

        @pl.when(s == NS - 1)
        def _():
            pltpu.sync_copy(_init_src(s * RPT0, RPT_LAST),
                            acc_sh.at[pl.ds(s * RPT0, RPT_LAST)])

        plsc.subcore_barrier()

        wait_sidx(0, 0)
        start_main(0, 0)

        def chunk_body(i, carry):
            b = lax.rem(i, 2)
            nb = 1 - b

            @pl.when(i + 1 < NCHUNK)
            def _():
                wait_sidx(i + 1, nb)
                start_main(i + 1, nb)

            wait_main(i, b)

            @pl.when(i + 2 < NCHUNK)
            def _():
                start_sidx(i + 2, b)

            rows_b = rows2.at[b]
            ea_b = ea2.at[b]

            @plsc.parallel_loop(0, C, unroll=4)
            def _(j):
                for k in range(D // L):
                    sl = (j, pl.ds(k * L, L))
                    rows_b[sl] = jnp.maximum(rows_b[sl] + ea_b[sl], 0.0)

            pltpu.sync_copy(rows2.at[b], acc_sh.at[dstb.at[b]], add=True)
            return carry

        lax.fori_loop(0, NCHUNK, chunk_body, 0)
        plsc.subcore_barrier()

        @pl.when(s < NS - 1)
        def _():
            pltpu.sync_copy(acc_sh.at[pl.ds(s * RPT0, RPT0)],
                            out_hbm.at[c, pl.ds(s * RPT0, RPT0)])

        @pl.when(s == NS - 1)
        def _():
            pltpu.sync_copy(acc_sh.at[pl.ds(s * RPT0, RPT_LAST)],
                            out_hbm.at[c, pl.ds(s * RPT0, RPT_LAST)])

    return sc_scatter


def _mlp_body(scale_ref, x_ref, p_ref, w1_ref, b1_ref, g_ref, bt_ref,
              w2_ref, b2_ref, out_ref):
    h = x_ref[...] * scale_ref[0] + p_ref[0] + p_ref[1]
    h = jnp.dot(h, w1_ref[...], preferred_element_type=jnp.float32) + b1_ref[...]
    mu = jnp.mean(h, axis=-1, keepdims=True)
    var = jnp.mean((h - mu) * (h - mu), axis=-1, keepdims=True)
    h = (h - mu) * lax.rsqrt(var + 1e-5) * g_ref[...] + bt_ref[...]
    h = jnp.maximum(h, 0.0)
    out_ref[...] = (
        jnp.dot(h, w2_ref[...], preferred_element_type=jnp.float32) + b2_ref[...]
    )


def _mlp(scale, x, partials, W1, b1, gamma, beta, W2, b2):
    N, D = x.shape
    H = W1.shape[1]
    BLK = 2000
    assert N % BLK == 0
    return pl.pallas_call(
        _mlp_body,
        grid=(N // BLK,),
        in_specs=[
            pl.BlockSpec(memory_space=pltpu.SMEM),
            pl.BlockSpec((BLK, D), lambda i: (i, 0)),
            pl.BlockSpec((NC, BLK, D), lambda i: (0, i, 0)),
            pl.BlockSpec((D, H), lambda i: (0, 0)),
            pl.BlockSpec((1, H), lambda i: (0, 0)),
            pl.BlockSpec((1, H), lambda i: (0, 0)),
            pl.BlockSpec((1, H), lambda i: (0, 0)),
            pl.BlockSpec((H, D), lambda i: (0, 0)),
            pl.BlockSpec((1, D), lambda i: (0, 0)),
        ],
        out_specs=pl.BlockSpec((BLK, D), lambda i: (i, 0)),
        out_shape=jax.ShapeDtypeStruct((N, D), jnp.float32),
    )(scale, x, partials, W1, b1.reshape(1, H), gamma.reshape(1, H),
      beta.reshape(1, H), W2, b2.reshape(1, D))


def kernel(x, edge_index, edge_attr, We, be, W1, b1, gamma, beta, W2, b2, eps):
    N, D = x.shape
    E = edge_attr.shape[0]

    NW = NC * NS
    C = 80
    NCH0, NCH1 = 62, 63
    E0 = NW * NCH0 * C
    assert E0 % 2560 == 0 and (E - E0) % 2560 == 0

    ea_t = edge_attr.T
    rea0 = _edge_mm(ea_t, We, be, 0, E0)
    rea1 = _edge_mm(ea_t, We, be, E0, E - E0)

    src = edge_index[1].astype(jnp.int32)
    dst = edge_index[0].astype(jnp.int32)
    src0 = src[:E0].reshape(NW, NCH0, C)
    dst0 = dst[:E0].reshape(NW, NCH0, C)
    src1 = src[E0:].reshape(NW, NCH1, C)
    dst1 = dst[E0:].reshape(NW, NCH1, C)
    zeros = jnp.zeros((N, D), dtype=jnp.float32)
    p0 = _make_sc_scatter(N, D, NCH0, False)(zeros, x, src0, dst0, rea0)
    partials = _make_sc_scatter(N, D, NCH1, True)(p0, x, src1, dst1, rea1)

    scale = (1.0 + eps).astype(jnp.float32).reshape(1)
    return _mlp(scale, x, partials, W1, b1, gamma, beta, W2, b2)


# speedup vs baseline: 1.5781x; 1.0305x over previous
import functools

import jax
import jax.numpy as jnp
from jax import lax
from jax.experimental import pallas as pl
from jax.experimental.pallas import tpu as pltpu
from jax.experimental.pallas import tpu_sc as plsc

NC = 2
NS = 16
L = 16


def _edge_mm_body(ea_ref, we_ref, be_ref, out_ref):
    out_ref[...] = lax.dot_general(
        ea_ref[...], we_ref[...], (((0,), (0,)), ((), ())),
        preferred_element_type=jnp.float32) + be_ref[...]


def _edge_mm(edge_attr_t, We, be, row0, nrows):
    ED = edge_attr_t.shape[0]
    D = We.shape[1]
    BLK = 2560
    assert row0 % BLK == 0 and nrows % BLK == 0
    blk0 = row0 // BLK
    return pl.pallas_call(
        _edge_mm_body,
        grid=(nrows // BLK,),
        in_specs=[
            pl.BlockSpec((ED, BLK), lambda i: (0, i + blk0)),
            pl.BlockSpec((ED, D), lambda i: (0, 0)),
            pl.BlockSpec((1, D), lambda i: (0, 0)),
        ],
        out_specs=pl.BlockSpec((BLK, D), lambda i: (i, 0)),
        out_shape=jax.ShapeDtypeStruct((nrows, D), jnp.float32),
    )(edge_attr_t, We, be.reshape(1, D))


def _make_sc_scatter(N, D, NCHUNK, chained):
    NW = NC * NS
    C = 80
    EPW = NCHUNK * C
    RPT0 = (N // NS) // 8 * 8
    RPT_LAST = N - (NS - 1) * RPT0
    assert RPT_LAST % 8 == 0
    SL2 = D // (2 * L)

    mesh = plsc.VectorSubcoreMesh(core_axis_name="c", subcore_axis_name="s")

    @functools.partial(
        pl.kernel,
        mesh=mesh,
        out_type=jax.ShapeDtypeStruct((NC, N, D), jnp.float32),
        scratch_types=[
            pltpu.VMEM((2, C), jnp.int32),
            pltpu.VMEM((2, C), jnp.int32),
            pltpu.VMEM((2, C, D), jnp.float32),
            pltpu.VMEM((2, C, D), jnp.float32),
            pltpu.VMEM_SHARED((N, D), jnp.float32),
            pltpu.SemaphoreType.DMA((2,)),
            pltpu.SemaphoreType.DMA((2,)),
            pltpu.SemaphoreType.DMA((2,)),
        ],
    )
    def sc_scatter(init_hbm, x_hbm, src_hbm, dst_hbm, rea_hbm, out_hbm,
                   srcb, dstb, rows2, ea2, acc_sh, gsem, esem, isem):
        c = lax.axis_index("c")
        s = lax.axis_index("s")
        w = c * NS + s
        base = w * EPW

        def start_sidx(j, b):
            pltpu.async_copy(src_hbm.at[w, j], srcb.at[b], isem.at[b])

        def wait_sidx(j, b):
            pltpu.make_async_copy(src_hbm.at[w, j], srcb.at[b],
                                  isem.at[b]).wait()

        def start_main(j, b):
            pltpu.async_copy(rea_hbm.at[pl.ds(base + j * C, C)], ea2.at[b],
                             esem.at[b])
            pltpu.async_copy(dst_hbm.at[w, j], dstb.at[b], esem.at[b])
            pltpu.async_copy(x_hbm.at[srcb.at[b]], rows2.at[b], gsem.at[b])

        def wait_main(j, b):
            pltpu.make_async_copy(rea_hbm.at[pl.ds(base + j * C, C)],
                                  ea2.at[b], esem.at[b]).wait()
            pltpu.make_async_copy(dst_hbm.at[w, j], dstb.at[b],
                                  esem.at[b]).wait()
            pltpu.make_async_copy(x_hbm.at[srcb.at[b]], rows2.at[b],
                                  gsem.at[b]).wait()

        start_sidx(0, 0)
        start_sidx(1, 1)

        def _init_src(r0, nr):
            if chained:
                return init_hbm.at[c, pl.ds(r0, nr)]
            return init_hbm.at[pl.ds(r0, nr)]

        @pl.when(s < NS - 1)
        def _():
            pltpu.sync_copy(_init_src(s * RPT0, RPT0),
                            acc_sh.at[pl.ds(s * RPT0, RPT0)])

        @pl.when(s == NS - 1)
        def _():
            pltpu.sync_copy(_init_src(s * RPT0, RPT_LAST),
                            acc_sh.at[pl.ds(s * RPT0, RPT_LAST)])

        plsc.subcore_barrier()

        wait_sidx(0, 0)
        start_main(0, 0)

        def chunk_body(i, carry):
            b = lax.rem(i, 2)
            nb = 1 - b

            @pl.when(i + 1 < NCHUNK)
            def _():
                wait_sidx(i + 1, nb)
                start_main(i + 1, nb)

            wait_main(i, b)

            @pl.when(i + 2 < NCHUNK)
            def _():
                start_sidx(i + 2, b)

            rows_b = rows2.at[b]
            ea_b = ea2.at[b]

            @plsc.parallel_loop(0, C, unroll=4)
            def _(j):
                for k in range(D // L):
                    sl = (j, pl.ds(k * L, L))
                    rows_b[sl] = jnp.maximum(rows_b[sl] + ea_b[sl], 0.0)

            pltpu.sync_copy(rows2.at[b], acc_sh.at[dstb.at[b]], add=True)
            return carry

        lax.fori_loop(0, NCHUNK, chunk_body, 0)
        plsc.subcore_barrier()

        @pl.when(s < NS - 1)
        def _():
            pltpu.sync_copy(acc_sh.at[pl.ds(s * RPT0, RPT0)],
                            out_hbm.at[c, pl.ds(s * RPT0, RPT0)])

        @pl.when(s == NS - 1)
        def _():
            pltpu.sync_copy(acc_sh.at[pl.ds(s * RPT0, RPT_LAST)],
                            out_hbm.at[c, pl.ds(s * RPT0, RPT_LAST)])

    return sc_scatter


def _mlp_body(scale_ref, x_ref, p_ref, w1_ref, b1_ref, g_ref, bt_ref,
              w2_ref, b2_ref, out_ref):
    h = x_ref[...] * scale_ref[0] + p_ref[0] + p_ref[1]
    h = jnp.dot(h, w1_ref[...], preferred_element_type=jnp.float32) + b1_ref[...]
    mu = jnp.mean(h, axis=-1, keepdims=True)
    var = jnp.mean((h - mu) * (h - mu), axis=-1, keepdims=True)
    h = (h - mu) * lax.rsqrt(var + 1e-5) * g_ref[...] + bt_ref[...]
    h = jnp.maximum(h, 0.0)
    out_ref[...] = (
        jnp.dot(h, w2_ref[...], preferred_element_type=jnp.float32) + b2_ref[...]
    )


def _mlp(scale, x, partials, W1, b1, gamma, beta, W2, b2):
    N, D = x.shape
    H = W1.shape[1]
    BLK = 2000
    assert N % BLK == 0
    return pl.pallas_call(
        _mlp_body,
        grid=(N // BLK,),
        in_specs=[
            pl.BlockSpec(memory_space=pltpu.SMEM),
            pl.BlockSpec((BLK, D), lambda i: (i, 0)),
            pl.BlockSpec((NC, BLK, D), lambda i: (0, i, 0)),
            pl.BlockSpec((D, H), lambda i: (0, 0)),
            pl.BlockSpec((1, H), lambda i: (0, 0)),
            pl.BlockSpec((1, H), lambda i: (0, 0)),
            pl.BlockSpec((1, H), lambda i: (0, 0)),
            pl.BlockSpec((H, D), lambda i: (0, 0)),
            pl.BlockSpec((1, D), lambda i: (0, 0)),
        ],
        out_specs=pl.BlockSpec((BLK, D), lambda i: (i, 0)),
        out_shape=jax.ShapeDtypeStruct((N, D), jnp.float32),
    )(scale, x, partials, W1, b1.reshape(1, H), gamma.reshape(1, H),
      beta.reshape(1, H), W2, b2.reshape(1, D))


def kernel(x, edge_index, edge_attr, We, be, W1, b1, gamma, beta, W2, b2, eps):
    N, D = x.shape
    E = edge_attr.shape[0]

    NW = NC * NS
    C = 80
    NCH0, NCH1 = 52, 73
    E0 = NW * NCH0 * C
    assert E0 % 2560 == 0 and (E - E0) % 2560 == 0

    ea_t = edge_attr.T
    rea0 = _edge_mm(ea_t, We, be, 0, E0)
    rea1 = _edge_mm(ea_t, We, be, E0, E - E0)

    src = edge_index[1].astype(jnp.int32)
    dst = edge_index[0].astype(jnp.int32)
    src0 = src[:E0].reshape(NW, NCH0, C)
    dst0 = dst[:E0].reshape(NW, NCH0, C)
    src1 = src[E0:].reshape(NW, NCH1, C)
    dst1 = dst[E0:].reshape(NW, NCH1, C)
    zeros = jnp.zeros((N, D), dtype=jnp.float32)
    p0 = _make_sc_scatter(N, D, NCH0, False)(zeros, x, src0, dst0, rea0)
    partials = _make_sc_scatter(N, D, NCH1, True)(p0, x, src1, dst1, rea1)

    scale = (1.0 + eps).astype(jnp.float32).reshape(1)
    return _mlp(scale, x, partials, W1, b1, gamma, beta, W2, b2)


# speedup vs baseline: 1.5789x; 1.0005x over previous
import functools

import jax
import jax.numpy as jnp
from jax import lax
from jax.experimental import pallas as pl
from jax.experimental.pallas import tpu as pltpu
from jax.experimental.pallas import tpu_sc as plsc

NC = 2
NS = 16
L = 16


def _edge_mm_body(ea_ref, we_ref, be_ref, out_ref):
    out_ref[...] = lax.dot_general(
        ea_ref[...], we_ref[...], (((0,), (0,)), ((), ())),
        preferred_element_type=jnp.float32) + be_ref[...]


def _edge_mm(edge_attr_t, We, be, row0, nrows):
    ED = edge_attr_t.shape[0]
    D = We.shape[1]
    BLK = 2560
    assert row0 % BLK == 0 and nrows % BLK == 0
    blk0 = row0 // BLK
    return pl.pallas_call(
        _edge_mm_body,
        grid=(nrows // BLK,),
        in_specs=[
            pl.BlockSpec((ED, BLK), lambda i: (0, i + blk0)),
            pl.BlockSpec((ED, D), lambda i: (0, 0)),
            pl.BlockSpec((1, D), lambda i: (0, 0)),
        ],
        out_specs=pl.BlockSpec((BLK, D), lambda i: (i, 0)),
        out_shape=jax.ShapeDtypeStruct((nrows, D), jnp.float32),
    )(edge_attr_t, We, be.reshape(1, D))


def _make_sc_scatter(N, D, NCHUNK, chained):
    NW = NC * NS
    C = 80
    EPW = NCHUNK * C
    RPT0 = (N // NS) // 8 * 8
    RPT_LAST = N - (NS - 1) * RPT0
    assert RPT_LAST % 8 == 0

    mesh = plsc.VectorSubcoreMesh(core_axis_name="c", subcore_axis_name="s")

    @functools.partial(
        pl.kernel,
        mesh=mesh,
        out_type=jax.ShapeDtypeStruct((NC, N, D), jnp.float32),
        scratch_types=[
            pltpu.VMEM((2, C), jnp.int32),
            pltpu.VMEM((2, C), jnp.int32),
            pltpu.VMEM((2, C, D), jnp.float32),
            pltpu.VMEM((2, C, D), jnp.float32),
            pltpu.VMEM_SHARED((N, D), jnp.float32),
            pltpu.SemaphoreType.DMA((2,)),
            pltpu.SemaphoreType.DMA((2,)),
            pltpu.SemaphoreType.DMA((2,)),
        ],
    )
    def sc_scatter(init_hbm, x_hbm, src_hbm, dst_hbm, rea_hbm, out_hbm,
                   srcb, dstb, rows2, ea2, acc_sh, gsem, esem, isem):
        c = lax.axis_index("c")
        s = lax.axis_index("s")
        w = c * NS + s
        base = w * EPW

        def start_sidx(j, b):
            pltpu.async_copy(src_hbm.at[w, j], srcb.at[b], isem.at[b])

        def wait_sidx(j, b):
            pltpu.make_async_copy(src_hbm.at[w, j], srcb.at[b],
                                  isem.at[b]).wait()

        def start_main(j, b):
            pltpu.async_copy(rea_hbm.at[pl.ds(base + j * C, C)], ea2.at[b],
                             esem.at[b])
            pltpu.async_copy(dst_hbm.at[w, j], dstb.at[b], esem.at[b])
            pltpu.async_copy(x_hbm.at[srcb.at[b]], rows2.at[b], gsem.at[b])

        def wait_main(j, b):
            pltpu.make_async_copy(rea_hbm.at[pl.ds(base + j * C, C)],
                                  ea2.at[b], esem.at[b]).wait()
            pltpu.make_async_copy(dst_hbm.at[w, j], dstb.at[b],
                                  esem.at[b]).wait()
            pltpu.make_async_copy(x_hbm.at[srcb.at[b]], rows2.at[b],
                                  gsem.at[b]).wait()

        start_sidx(0, 0)
        start_sidx(1, 1)

        def _init_src(r0, nr):
            if chained:
                return init_hbm.at[c, pl.ds(r0, nr)]
            return init_hbm.at[pl.ds(r0, nr)]

        @pl.when(s < NS - 1)
        def _():
            pltpu.sync_copy(_init_src(s * RPT0, RPT0),
                            acc_sh.at[pl.ds(s * RPT0, RPT0)])

        @pl.when(s == NS - 1)
        def _():
            pltpu.sync_copy(_init_src(s * RPT0, RPT_LAST),
                            acc_sh.at[pl.ds(s * RPT0, RPT_LAST)])

        plsc.subcore_barrier()

        wait_sidx(0, 0)
        start_main(0, 0)

        def chunk_body(i, carry):
            b = lax.rem(i, 2)
            nb = 1 - b

            @pl.when(i + 1 < NCHUNK)
            def _():
                wait_sidx(i + 1, nb)
                start_main(i + 1, nb)

            wait_main(i, b)

            @pl.when(i + 2 < NCHUNK)
            def _():
                start_sidx(i + 2, b)

            rows_b = rows2.at[b]
            ea_b = ea2.at[b]

            @plsc.parallel_loop(0, C, unroll=4)
            def _(j):
                for k in range(D // L):
                    sl = (j, pl.ds(k * L, L))
                    rows_b[sl] = jnp.maximum(rows_b[sl] + ea_b[sl], 0.0)

            pltpu.sync_copy(rows2.at[b], acc_sh.at[dstb.at[b]], add=True)
            return carry

        lax.fori_loop(0, NCHUNK, chunk_body, 0)
        plsc.subcore_barrier()

        @pl.when(s < NS - 1)
        def _():
            pltpu.sync_copy(acc_sh.at[pl.ds(s * RPT0, RPT0)],
                            out_hbm.at[c, pl.ds(s * RPT0, RPT0)])

        @pl.when(s == NS - 1)
        def _():
            pltpu.sync_copy(acc_sh.at[pl.ds(s * RPT0, RPT_LAST)],
                            out_hbm.at[c, pl.ds(s * RPT0, RPT_LAST)])

    return sc_scatter


def _mlp_body(scale_ref, x_ref, p_ref, w1_ref, b1_ref, g_ref, bt_ref,
              w2_ref, b2_ref, out_ref):
    h = x_ref[...] * scale_ref[0] + p_ref[0] + p_ref[1]
    h = jnp.dot(h, w1_ref[...], preferred_element_type=jnp.float32) + b1_ref[...]
    mu = jnp.mean(h, axis=-1, keepdims=True)
    var = jnp.mean((h - mu) * (h - mu), axis=-1, keepdims=True)
    h = (h - mu) * lax.rsqrt(var + 1e-5) * g_ref[...] + bt_ref[...]
    h = jnp.maximum(h, 0.0)
    out_ref[...] = (
        jnp.dot(h, w2_ref[...], preferred_element_type=jnp.float32) + b2_ref[...]
    )


def _mlp(scale, x, partials, W1, b1, gamma, beta, W2, b2):
    N, D = x.shape
    H = W1.shape[1]
    BLK = 2000
    assert N % BLK == 0
    return pl.pallas_call(
        _mlp_body,
        grid=(N // BLK,),
        in_specs=[
            pl.BlockSpec(memory_space=pltpu.SMEM),
            pl.BlockSpec((BLK, D), lambda i: (i, 0)),
            pl.BlockSpec((NC, BLK, D), lambda i: (0, i, 0)),
            pl.BlockSpec((D, H), lambda i: (0, 0)),
            pl.BlockSpec((1, H), lambda i: (0, 0)),
            pl.BlockSpec((1, H), lambda i: (0, 0)),
            pl.BlockSpec((1, H), lambda i: (0, 0)),
            pl.BlockSpec((H, D), lambda i: (0, 0)),
            pl.BlockSpec((1, D), lambda i: (0, 0)),
        ],
        out_specs=pl.BlockSpec((BLK, D), lambda i: (i, 0)),
        out_shape=jax.ShapeDtypeStruct((N, D), jnp.float32),
    )(scale, x, partials, W1, b1.reshape(1, H), gamma.reshape(1, H),
      beta.reshape(1, H), W2, b2.reshape(1, D))


def kernel(x, edge_index, edge_attr, We, be, W1, b1, gamma, beta, W2, b2, eps):
    N, D = x.shape
    E = edge_attr.shape[0]

    NW = NC * NS
    C = 80
    NCH0, NCH1 = 52, 73
    E0 = NW * NCH0 * C
    assert E0 % 2560 == 0 and (E - E0) % 2560 == 0

    ea_t = edge_attr.T
    rea0 = _edge_mm(ea_t, We, be, 0, E0)
    rea1 = _edge_mm(ea_t, We, be, E0, E - E0)

    src = edge_index[1].astype(jnp.int32)
    dst = edge_index[0].astype(jnp.int32)
    src0 = src[:E0].reshape(NW, NCH0, C)
    dst0 = dst[:E0].reshape(NW, NCH0, C)
    src1 = src[E0:].reshape(NW, NCH1, C)
    dst1 = dst[E0:].reshape(NW, NCH1, C)
    zeros = jnp.zeros((N, D), dtype=jnp.float32)
    p0 = _make_sc_scatter(N, D, NCH0, False)(zeros, x, src0, dst0, rea0)
    partials = _make_sc_scatter(N, D, NCH1, True)(p0, x, src1, dst1, rea1)

    scale = (1.0 + eps).astype(jnp.float32).reshape(1)
    return _mlp(scale, x, partials, W1, b1, gamma, beta, W2, b2)
